# baseline (speedup 1.0000x reference)
import functools

import jax
import jax.numpy as jnp
from jax import lax
from jax.experimental import pallas as pl
from jax.experimental.pallas import tpu as pltpu
from jax.experimental.pallas import tpu_sc as plsc

N = 10000
E = 320000
D = 128
DE = 16

NC = 2
NS = 16
NW = NC * NS
NPAD = 10240
TROWS = NPAD // NS
SB = 128
NG = E // SB
KC = 8
NCHT = (NG + KC - 1) // KC
NCHUNK = NCHT // NW + 1
NBUF = 2
REMC = NG % KC


def _winner_update(win_ref, c16, eid16):
    io = lax.iota(jnp.int32, 16)
    dn = lax.GatherDimensionNumbers(
        offset_dims=(), collapsed_slice_dims=(0,), start_index_map=(0,))
    dup = io < 0
    for k in range(1, 16):
        sh = lax.gather(
            c16, jnp.minimum(io + k, 15)[:, None], dimension_numbers=dn,
            slice_sizes=(1,), mode=lax.GatherScatterMode.PROMISE_IN_BOUNDS)
        dup = dup | ((sh == c16) & (io < 16 - k))
    plsc.store_scatter(win_ref, [c16], eid16, mask=jnp.logical_not(dup))


def _sc_segment(x, row2d, col2d):
    mesh = plsc.VectorSubcoreMesh(core_axis_name="c", subcore_axis_name="s")

    @functools.partial(
        pl.kernel,
        mesh=mesh,
        out_type=[
            jax.ShapeDtypeStruct((NC * NPAD, D), jnp.float32),
            jax.ShapeDtypeStruct((NC * NPAD,), jnp.float32),
            jax.ShapeDtypeStruct((NW * NPAD,), jnp.int32),
        ],
        scratch_types=[
            pltpu.VMEM_SHARED((NPAD, D), jnp.float32),
            pltpu.VMEM_SHARED((NPAD,), jnp.float32),
            pltpu.VMEM((KC, SB), jnp.int32),
            pltpu.VMEM((KC, SB), jnp.int32),
            pltpu.VMEM((SB, D), jnp.float32),
            pltpu.VMEM((SB, D), jnp.float32),
            pltpu.VMEM((SB,), jnp.float32),
            pltpu.VMEM((NPAD,), jnp.int32),
            pltpu.VMEM((TROWS,), jnp.float32),
            pltpu.SemaphoreType.DMA,
            pltpu.SemaphoreType.DMA,
        ],
        compiler_params=pltpu.CompilerParams(needs_layout_passes=False),
    )
    def sc_a(x_hbm, row_hbm, col_hbm, sums_hbm, cnts_hbm, win32_hbm,
             accum, cnt_sh, rowc, colc, rb0, rb1,
             ones, win_t, zbuf, sem0, sem1):
        c = lax.axis_index("c")
        s = lax.axis_index("s")
        wid = c * NS + s
        rbufs = [rb0, rb1]
        sems = [sem0, sem1]
        zero16 = jnp.zeros((16,), jnp.float32)

        def zrow(r, _):
            for j in range(D // 16):
                rb0[r, pl.ds(j * 16, 16)] = zero16
            return 0
        lax.fori_loop(0, SB, zrow, 0)

        def zsmall(i, _):
            ones[pl.ds(i * 16, 16)] = zero16 + 1.0
            return 0
        lax.fori_loop(0, SB // 16, zsmall, 0)

        def zzb(i, _):
            zbuf[pl.ds(i * 16, 16)] = zero16
            return 0
        lax.fori_loop(0, TROWS // 16, zzb, 0)

        neg1 = jnp.full((16,), -1, jnp.int32)

        def zwin(i, _):
            win_t[pl.ds(i * 16, 16)] = neg1
            return 0
        lax.fori_loop(0, NPAD // 16, zwin, 0)

        nbase = s * TROWS
        for q in range(TROWS // SB):
            pltpu.sync_copy(rb0, accum.at[pl.ds(nbase + q * SB, SB)])
        pltpu.sync_copy(zbuf, cnt_sh.at[pl.ds(nbase, TROWS)])
        plsc.subcore_barrier()

        cs = wid * NCHT // NW
        cntc = (wid + 1) * NCHT // NW - cs

        def chunk(ch, _):
            @pl.when(ch < cntc)
            def _():
                gc = pl.multiple_of((cs + ch) * KC, 8)

                @pl.when(gc + KC <= NG)
                def _():
                    pltpu.sync_copy(row_hbm.at[pl.ds(gc, KC)], rowc)
                    pltpu.sync_copy(col_hbm.at[pl.ds(gc, KC)], colc)

                @pl.when(gc + KC > NG)
                def _():
                    pltpu.sync_copy(row_hbm.at[pl.ds(gc, REMC)],
                                    rowc.at[pl.ds(0, REMC)])
                    pltpu.sync_copy(col_hbm.at[pl.ds(gc, REMC)],
                                    colc.at[pl.ds(0, REMC)])
                for b in range(NBUF):
                    @pl.when(gc + b < NG)
                    def _():
                        pltpu.async_copy(x_hbm.at[rowc.at[b]],
                                         rbufs[b], sems[b])
                for b in range(KC):
                    r = b % NBUF

                    @pl.when(gc + b < NG)
                    def _():
                        pltpu.make_async_copy(x_hbm.at[rowc.at[b]],
                                              rbufs[r], sems[r]).wait()
                        pltpu.sync_copy(rbufs[r], accum.at[colc.at[b]],
                                        add=True)
                        pltpu.sync_copy(ones, cnt_sh.at[colc.at[b]],
                                        add=True)
                        ebase = (gc + b) * SB
                        for v in range(SB // 16):
                            c16 = colc[b, pl.ds(v * 16, 16)]
                            eid16 = ebase + v * 16 + lax.iota(jnp.int32, 16)
                            _winner_update(win_t, c16, eid16)
                        if b + NBUF < KC:
                            @pl.when(gc + b + NBUF < NG)
                            def _():
                                pltpu.async_copy(
                                    x_hbm.at[rowc.at[b + NBUF]],
                                    rbufs[r], sems[r])
            return 0
        lax.fori_loop(0, NCHUNK, chunk, 0)

        plsc.subcore_barrier()
        pltpu.sync_copy(win_t,
                        win32_hbm.at[pl.ds(pl.multiple_of(wid * NPAD, 8),
                                           NPAD)])
        hb = pl.multiple_of(c * NPAD + nbase, 8)
        pltpu.sync_copy(accum.at[pl.ds(nbase, TROWS)],
                        sums_hbm.at[pl.ds(hb, TROWS)])
        pltpu.sync_copy(cnt_sh.at[pl.ds(nbase, TROWS)],
                        cnts_hbm.at[pl.ds(hb, TROWS)])

    return sc_a(x, row2d, col2d)


def _tc_winmax(win32):
    def body(w_ref, o_ref):
        o_ref[...] = jnp.max(w_ref[...], axis=0, keepdims=True)

    return pl.pallas_call(
        body,
        out_shape=jax.ShapeDtypeStruct((1, NPAD), jnp.int32),
    )(win32)


def _sc_winner(winf, ea_wide):
    mesh = plsc.VectorSubcoreMesh(core_axis_name="c", subcore_axis_name="s")
    NB = NPAD // SB

    @functools.partial(
        pl.kernel,
        mesh=mesh,
        out_type=jax.ShapeDtypeStruct((NPAD, D), jnp.float32),
        scratch_types=[
            pltpu.VMEM((SB,), jnp.int32),
            pltpu.VMEM((SB,), jnp.int32),
            pltpu.VMEM((SB, D), jnp.float32),
            pltpu.SemaphoreType.DMA,
        ],
        compiler_params=pltpu.CompilerParams(needs_layout_passes=False),
    )
    def sc_b(win_hbm, ea_hbm, eaw_hbm, w0, idxb, rowsb, sem):
        c = lax.axis_index("c")
        s = lax.axis_index("s")
        wid = c * NS + s
        bst = NB * wid // NW
        bcnt = NB * (wid + 1) // NW - bst
        for t in range(3):
            b = bst + t

            @pl.when(t < bcnt)
            def _():
                nb = pl.multiple_of(b * SB, 8)
                pltpu.sync_copy(win_hbm.at[pl.ds(nb, SB)], w0)

                def mx(v, _):
                    sl = pl.ds(v * 16, 16)
                    cl = jnp.clip(w0[sl], 0, E - 1)
                    idxb[sl] = lax.shift_right_logical(cl, 3)
                    return 0
                lax.fori_loop(0, SB // 16, mx, 0)
                pltpu.async_copy(ea_hbm.at[idxb], rowsb, sem).wait()
                pltpu.sync_copy(rowsb, eaw_hbm.at[pl.ds(nb, SB)])

    return sc_b(winf, ea_wide)


BK = 1024
NBLK = NPAD // BK


def _tc_dense(sa, sb, ca, cb, win, eaw, x, W_l, b_l, W_r, edge_W, edge_b,
              att_W, att_b, bn_gamma, bn_beta):
    dn = (((1,), (1,)), ((), ()))

    def body1(sa_ref, sb_ref, ca_ref, cb_ref, win_ref, eaw_ref, x_ref,
              wl_ref, bl_ref, wr_ref, ew_ref, eb_ref, aw_ref, ab_ref,
              pre_ref, ps_ref, pq_ref):
        i = pl.program_id(0)
        summed = sa_ref[...] + sb_ref[...]
        counts = ca_ref[...] + cb_ref[...]
        mean = summed / jnp.clip(counts, 1.0, None)
        out = (lax.dot_general(mean, wl_ref[...], dn,
                               preferred_element_type=jnp.float32)
               + lax.dot_general(x_ref[...], wr_ref[...], dn,
                                 preferred_element_type=jnp.float32)
               + bl_ref[...])
        winv = win_ref[...]
        woff = jnp.clip(winv, 0, E - 1) & 7
        ea = jnp.zeros((BK, DE), jnp.float32)
        for k in range(8):
            ea = ea + jnp.where(woff == k,
                                eaw_ref[:, k * DE:(k + 1) * DE], 0.0)
        edge_t = (lax.dot_general(ea, ew_ref[...], dn,
                                  preferred_element_type=jnp.float32)
                  + eb_ref[...])
        a1 = aw_ref[:, 0:D]
        a2 = aw_ref[:, D:2 * D]
        logit = (lax.dot_general(out, a1, dn,
                                 preferred_element_type=jnp.float32)
                 + lax.dot_general(edge_t, a2, dn,
                                   preferred_element_type=jnp.float32)
                 + ab_ref[...])
        att = jax.nn.sigmoid(logit)
        out = out + jnp.where(winv >= 0, att * edge_t, 0.0)
        pre_ref[...] = out
        gid = i * BK + lax.broadcasted_iota(jnp.int32, (BK, 1), 0)
        outm = jnp.where(gid < N, out, 0.0)
        ps_ref[...] = jnp.sum(outm, axis=0, keepdims=True).reshape(1, 1, D)
        pq_ref[...] = jnp.sum(outm * outm, axis=0,
                              keepdims=True).reshape(1, 1, D)

    row_blk = pl.BlockSpec((BK, D), lambda i: (i, 0))
    col_blk = pl.BlockSpec((BK, 1), lambda i: (i, 0))
    full = lambda r, c: pl.BlockSpec((r, c), lambda i: (0, 0))
    pre, ps, pq = pl.pallas_call(
        body1,
        grid=(NBLK,),
        in_specs=[row_blk, row_blk, col_blk, col_blk, col_blk, row_blk,
                  row_blk, full(D, D), full(1, D), full(D, D), full(D, DE),
                  full(1, D), full(1, 2 * D), full(1, 1)],
        out_specs=[row_blk, pl.BlockSpec((1, 1, D), lambda i: (i, 0, 0)),
                   pl.BlockSpec((1, 1, D), lambda i: (i, 0, 0))],
        out_shape=[jax.ShapeDtypeStruct((NPAD, D), jnp.float32),
                   jax.ShapeDtypeStruct((NBLK, 1, D), jnp.float32),
                   jax.ShapeDtypeStruct((NBLK, 1, D), jnp.float32)],
    )(sa, sb, ca, cb, win, eaw, x, W_l, b_l, W_r, edge_W, edge_b,
      att_W, att_b)

    def body2(pre_ref, ps_ref, pq_ref, g_ref, be_ref, o_ref):
        s = jnp.sum(ps_ref[...], axis=0)
        q = jnp.sum(pq_ref[...], axis=0)
        mu = s * (1.0 / N)
        var = q * (1.0 / N) - mu * mu
        out = ((pre_ref[...] - mu) * lax.rsqrt(var + 1e-5) * g_ref[...]
               + be_ref[...])
        o_ref[...] = jnp.maximum(out + out, 0.0)

    return pl.pallas_call(
        body2,
        grid=(NBLK,),
        in_specs=[row_blk,
                  pl.BlockSpec((NBLK, 1, D), lambda i: (0, 0, 0)),
                  pl.BlockSpec((NBLK, 1, D), lambda i: (0, 0, 0)),
                  full(1, D), full(1, D)],
        out_specs=row_blk,
        out_shape=jax.ShapeDtypeStruct((NPAD, D), jnp.float32),
    )(pre, ps, pq, bn_gamma, bn_beta)


def kernel(x, edge_index, edge_attr, W_l, b_l, W_r, edge_W, edge_b,
           att_W, att_b, bn_gamma, bn_beta):
    row = edge_index[0]
    col = edge_index[1]
    sums2, cnts2, win32 = _sc_segment(x, row.reshape(NG, SB),
                                      col.reshape(NG, SB))
    winf = _tc_winmax(win32.reshape(NW, NPAD)).reshape(NPAD)
    eaw = _sc_winner(winf, edge_attr.reshape(E // 8, 8 * DE))
    xpad = jnp.pad(x, ((0, NPAD - N), (0, 0)))
    out = _tc_dense(
        sums2[:NPAD],
        sums2[NPAD:],
        cnts2[:NPAD].reshape(NPAD, 1),
        cnts2[NPAD:].reshape(NPAD, 1),
        winf.reshape(NPAD, 1),
        eaw,
        xpad,
        W_l,
        b_l.reshape(1, D),
        W_r,
        edge_W,
        edge_b.reshape(1, D),
        att_W,
        att_b.reshape(1, 1),
        bn_gamma.reshape(1, D),
        bn_beta.reshape(1, D),
    )
    return out[:N]


# speedup vs baseline: 1.0140x; 1.0140x over previous
import functools

import jax
import jax.numpy as jnp
from jax import lax
from jax.experimental import pallas as pl
from jax.experimental.pallas import tpu as pltpu
from jax.experimental.pallas import tpu_sc as plsc

N = 10000
E = 320000
D = 128
DE = 16

NC = 2
NS = 16
NW = NC * NS
NPAD = 10240
TROWS = NPAD // NS
SB = 128
NG = E // SB
KC = 8
NCHT = (NG + KC - 1) // KC
NCHUNK = NCHT // NW + 1
NBUF = 2
REMC = NG % KC


def _winner_update(win_ref, c16, eid16):
    io = lax.iota(jnp.int32, 16)
    dn = lax.GatherDimensionNumbers(
        offset_dims=(), collapsed_slice_dims=(0,), start_index_map=(0,))
    dup = io < 0
    for k in range(1, 16):
        sh = lax.gather(
            c16, jnp.minimum(io + k, 15)[:, None], dimension_numbers=dn,
            slice_sizes=(1,), mode=lax.GatherScatterMode.PROMISE_IN_BOUNDS)
        dup = dup | ((sh == c16) & (io < 16 - k))
    plsc.store_scatter(win_ref, [c16], eid16, mask=jnp.logical_not(dup))


def _sc_segment(x, row2d, col2d):
    mesh = plsc.VectorSubcoreMesh(core_axis_name="c", subcore_axis_name="s")

    @functools.partial(
        pl.kernel,
        mesh=mesh,
        out_type=[
            jax.ShapeDtypeStruct((NC * NPAD, D), jnp.float32),
            jax.ShapeDtypeStruct((NC * NPAD,), jnp.float32),
            jax.ShapeDtypeStruct((NW * NPAD,), jnp.int32),
        ],
        scratch_types=[
            pltpu.VMEM_SHARED((NPAD, D), jnp.float32),
            pltpu.VMEM_SHARED((NPAD,), jnp.float32),
            pltpu.VMEM((KC, SB), jnp.int32),
            pltpu.VMEM((KC, SB), jnp.int32),
            pltpu.VMEM((SB, D), jnp.float32),
            pltpu.VMEM((SB, D), jnp.float32),
            pltpu.VMEM((SB,), jnp.float32),
            pltpu.VMEM((NPAD,), jnp.int32),
            pltpu.VMEM((TROWS,), jnp.float32),
            pltpu.SemaphoreType.DMA,
            pltpu.SemaphoreType.DMA,
        ],
        compiler_params=pltpu.CompilerParams(needs_layout_passes=False),
    )
    def sc_a(x_hbm, row_hbm, col_hbm, sums_hbm, cnts_hbm, win32_hbm,
             accum, cnt_sh, rowc, colc, rb0, rb1,
             ones, win_t, zbuf, sem0, sem1):
        c = lax.axis_index("c")
        s = lax.axis_index("s")
        wid = c * NS + s
        rbufs = [rb0, rb1]
        sems = [sem0, sem1]
        zero16 = jnp.zeros((16,), jnp.float32)

        def zrow(r, _):
            for j in range(D // 16):
                rb0[r, pl.ds(j * 16, 16)] = zero16
            return 0
        lax.fori_loop(0, SB, zrow, 0)

        def zsmall(i, _):
            ones[pl.ds(i * 16, 16)] = zero16 + 1.0
            return 0
        lax.fori_loop(0, SB // 16, zsmall, 0)

        def zzb(i, _):
            zbuf[pl.ds(i * 16, 16)] = zero16
            return 0
        lax.fori_loop(0, TROWS // 16, zzb, 0)

        neg1 = jnp.full((16,), -1, jnp.int32)

        def zwin(i, _):
            win_t[pl.ds(i * 16, 16)] = neg1
            return 0
        lax.fori_loop(0, NPAD // 16, zwin, 0)

        nbase = s * TROWS
        for q in range(TROWS // SB):
            pltpu.sync_copy(rb0, accum.at[pl.ds(nbase + q * SB, SB)])
        pltpu.sync_copy(zbuf, cnt_sh.at[pl.ds(nbase, TROWS)])
        plsc.subcore_barrier()

        cs = wid * NCHT // NW
        cntc = (wid + 1) * NCHT // NW - cs

        def chunk(ch, _):
            @pl.when(ch < cntc)
            def _():
                gc = pl.multiple_of((cs + ch) * KC, 8)

                @pl.when(gc + KC <= NG)
                def _():
                    pltpu.sync_copy(row_hbm.at[pl.ds(gc, KC)], rowc)
                    pltpu.sync_copy(col_hbm.at[pl.ds(gc, KC)], colc)

                @pl.when(gc + KC > NG)
                def _():
                    pltpu.sync_copy(row_hbm.at[pl.ds(gc, REMC)],
                                    rowc.at[pl.ds(0, REMC)])
                    pltpu.sync_copy(col_hbm.at[pl.ds(gc, REMC)],
                                    colc.at[pl.ds(0, REMC)])
                for b in range(NBUF):
                    @pl.when(gc + b < NG)
                    def _():
                        pltpu.async_copy(x_hbm.at[rowc.at[b]],
                                         rbufs[b], sems[b])
                for b in range(KC):
                    r = b % NBUF

                    @pl.when(gc + b < NG)
                    def _():
                        pltpu.make_async_copy(x_hbm.at[rowc.at[b]],
                                              rbufs[r], sems[r]).wait()
                        pltpu.sync_copy(rbufs[r], accum.at[colc.at[b]],
                                        add=True)
                        pltpu.sync_copy(ones, cnt_sh.at[colc.at[b]],
                                        add=True)
                        ebase = (gc + b) * SB
                        for v in range(SB // 16):
                            c16 = colc[b, pl.ds(v * 16, 16)]
                            eid16 = ebase + v * 16 + lax.iota(jnp.int32, 16)
                            _winner_update(win_t, c16, eid16)
                        if b + NBUF < KC:
                            @pl.when(gc + b + NBUF < NG)
                            def _():
                                pltpu.async_copy(
                                    x_hbm.at[rowc.at[b + NBUF]],
                                    rbufs[r], sems[r])
            return 0
        lax.fori_loop(0, NCHUNK, chunk, 0)

        plsc.subcore_barrier()
        pltpu.sync_copy(win_t,
                        win32_hbm.at[pl.ds(pl.multiple_of(wid * NPAD, 8),
                                           NPAD)])
        hb = pl.multiple_of(c * NPAD + nbase, 8)
        pltpu.sync_copy(accum.at[pl.ds(nbase, TROWS)],
                        sums_hbm.at[pl.ds(hb, TROWS)])
        pltpu.sync_copy(cnt_sh.at[pl.ds(nbase, TROWS)],
                        cnts_hbm.at[pl.ds(hb, TROWS)])

    return sc_a(x, row2d, col2d)


def _tc_winmax(win32):
    def body(w_ref, o_ref):
        o_ref[...] = jnp.max(w_ref[...], axis=0, keepdims=True)

    return pl.pallas_call(
        body,
        out_shape=jax.ShapeDtypeStruct((1, NPAD), jnp.int32),
    )(win32)


def _sc_winner(winf, ea_wide):
    mesh = plsc.VectorSubcoreMesh(core_axis_name="c", subcore_axis_name="s")
    NB = NPAD // SB

    @functools.partial(
        pl.kernel,
        mesh=mesh,
        out_type=jax.ShapeDtypeStruct((NPAD, D), jnp.float32),
        scratch_types=[
            pltpu.VMEM((SB,), jnp.int32),
            pltpu.VMEM((SB,), jnp.int32),
            pltpu.VMEM((SB, D), jnp.float32),
            pltpu.SemaphoreType.DMA,
        ],
        compiler_params=pltpu.CompilerParams(needs_layout_passes=False),
    )
    def sc_b(win_hbm, ea_hbm, eaw_hbm, w0, idxb, rowsb, sem):
        c = lax.axis_index("c")
        s = lax.axis_index("s")
        wid = c * NS + s
        bst = NB * wid // NW
        bcnt = NB * (wid + 1) // NW - bst
        for t in range(3):
            b = bst + t

            @pl.when(t < bcnt)
            def _():
                nb = pl.multiple_of(b * SB, 8)
                pltpu.sync_copy(win_hbm.at[pl.ds(nb, SB)], w0)

                def mx(v, _):
                    sl = pl.ds(v * 16, 16)
                    cl = jnp.clip(w0[sl], 0, E - 1)
                    idxb[sl] = lax.shift_right_logical(cl, 3)
                    return 0
                lax.fori_loop(0, SB // 16, mx, 0)
                pltpu.async_copy(ea_hbm.at[idxb], rowsb, sem).wait()
                pltpu.sync_copy(rowsb, eaw_hbm.at[pl.ds(nb, SB)])

    return sc_b(winf, ea_wide)


BK = 1024
NBLK = NPAD // BK


def _tc_dense(sa, sb, ca, cb, win, eaw, x, W_l, b_l, W_r, edge_W, edge_b,
              att_W, att_b, bn_gamma, bn_beta):
    dn = (((1,), (1,)), ((), ()))

    def body1(sa_ref, sb_ref, ca_ref, cb_ref, win_ref, eaw_ref, x_ref,
              wl_ref, bl_ref, wr_ref, ew_ref, eb_ref, aw_ref, ab_ref,
              pre_ref, ps_ref, pq_ref):
        i = pl.program_id(0)
        summed = sa_ref[...] + sb_ref[...]
        counts = ca_ref[...] + cb_ref[...]
        mean = summed / jnp.clip(counts, 1.0, None)
        out = (lax.dot_general(mean, wl_ref[...], dn,
                               preferred_element_type=jnp.float32)
               + lax.dot_general(x_ref[...], wr_ref[...], dn,
                                 preferred_element_type=jnp.float32)
               + bl_ref[...])
        winv = win_ref[...]
        woff = jnp.clip(winv, 0, E - 1) & 7
        ea = jnp.zeros((BK, DE), jnp.float32)
        for k in range(8):
            ea = ea + jnp.where(woff == k,
                                eaw_ref[:, k * DE:(k + 1) * DE], 0.0)
        edge_t = (lax.dot_general(ea, ew_ref[...], dn,
                                  preferred_element_type=jnp.float32)
                  + eb_ref[...])
        a1 = aw_ref[:, 0:D]
        a2 = aw_ref[:, D:2 * D]
        logit = (lax.dot_general(out, a1, dn,
                                 preferred_element_type=jnp.float32)
                 + lax.dot_general(edge_t, a2, dn,
                                   preferred_element_type=jnp.float32)
                 + ab_ref[...])
        att = jax.nn.sigmoid(logit)
        out = out + jnp.where(winv >= 0, att * edge_t, 0.0)
        pre_ref[...] = out
        gid = i * BK + lax.broadcasted_iota(jnp.int32, (BK, 1), 0)
        outm = jnp.where(gid < N, out, 0.0)
        ps_ref[...] = jnp.sum(outm, axis=0, keepdims=True).reshape(1, 1, D)
        pq_ref[...] = jnp.sum(outm * outm, axis=0,
                              keepdims=True).reshape(1, 1, D)

    row_blk = pl.BlockSpec((BK, D), lambda i: (i, 0))
    col_blk = pl.BlockSpec((BK, 1), lambda i: (i, 0))
    row_blk2 = pl.BlockSpec((BK, D), lambda i: (i + NBLK, 0))
    col_blk2 = pl.BlockSpec((BK, 1), lambda i: (i + NBLK, 0))
    full = lambda r, c: pl.BlockSpec((r, c), lambda i: (0, 0))
    pre, ps, pq = pl.pallas_call(
        body1,
        grid=(NBLK,),
        in_specs=[row_blk, row_blk2, col_blk, col_blk2, col_blk, row_blk,
                  row_blk, full(D, D), full(1, D), full(D, D), full(D, DE),
                  full(1, D), full(1, 2 * D), full(1, 1)],
        out_specs=[row_blk, pl.BlockSpec((1, 1, D), lambda i: (i, 0, 0)),
                   pl.BlockSpec((1, 1, D), lambda i: (i, 0, 0))],
        out_shape=[jax.ShapeDtypeStruct((NPAD, D), jnp.float32),
                   jax.ShapeDtypeStruct((NBLK, 1, D), jnp.float32),
                   jax.ShapeDtypeStruct((NBLK, 1, D), jnp.float32)],
    )(sa, sb, ca, cb, win, eaw, x, W_l, b_l, W_r, edge_W, edge_b,
      att_W, att_b)

    def body2(pre_ref, ps_ref, pq_ref, g_ref, be_ref, o_ref):
        s = jnp.sum(ps_ref[...], axis=0)
        q = jnp.sum(pq_ref[...], axis=0)
        mu = s * (1.0 / N)
        var = q * (1.0 / N) - mu * mu
        out = ((pre_ref[...] - mu) * lax.rsqrt(var + 1e-5) * g_ref[...]
               + be_ref[...])
        o_ref[...] = jnp.maximum(out + out, 0.0)

    return pl.pallas_call(
        body2,
        grid=(NBLK,),
        in_specs=[row_blk,
                  pl.BlockSpec((NBLK, 1, D), lambda i: (0, 0, 0)),
                  pl.BlockSpec((NBLK, 1, D), lambda i: (0, 0, 0)),
                  full(1, D), full(1, D)],
        out_specs=row_blk,
        out_shape=jax.ShapeDtypeStruct((NPAD, D), jnp.float32),
    )(pre, ps, pq, bn_gamma, bn_beta)


def kernel(x, edge_index, edge_attr, W_l, b_l, W_r, edge_W, edge_b,
           att_W, att_b, bn_gamma, bn_beta):
    row = edge_index[0]
    col = edge_index[1]
    sums2, cnts2, win32 = _sc_segment(x, row.reshape(NG, SB),
                                      col.reshape(NG, SB))
    winf = _tc_winmax(win32.reshape(NW, NPAD)).reshape(NPAD)
    eaw = _sc_winner(winf, edge_attr.reshape(E // 8, 8 * DE))
    xpad = jnp.pad(x, ((0, NPAD - N), (0, 0)))
    cnts2d = cnts2.reshape(NC * NPAD, 1)
    out = _tc_dense(
        sums2,
        sums2,
        cnts2d,
        cnts2d,
        winf.reshape(NPAD, 1),
        eaw,
        xpad,
        W_l,
        b_l.reshape(1, D),
        W_r,
        edge_W,
        edge_b.reshape(1, D),
        att_W,
        att_b.reshape(1, 1),
        bn_gamma.reshape(1, D),
        bn_beta.reshape(1, D),
    )
    return out[:N]


# speedup vs baseline: 1.0228x; 1.0087x over previous
import functools

import jax
import jax.numpy as jnp
from jax import lax
from jax.experimental import pallas as pl
from jax.experimental.pallas import tpu as pltpu
from jax.experimental.pallas import tpu_sc as plsc

N = 10000
E = 320000
D = 128
DE = 16

NC = 2
NS = 16
NW = NC * NS
NPAD = 10240
TROWS = NPAD // NS
SB = 128
NG = E // SB
KC = 8
NCHT = (NG + KC - 1) // KC
NCHUNK = NCHT // NW + 1
NBUF = 2
REMC = NG % KC


def _winner_update(win_ref, c16, eid16):
    io = lax.iota(jnp.int32, 16)
    dn = lax.GatherDimensionNumbers(
        offset_dims=(), collapsed_slice_dims=(0,), start_index_map=(0,))
    dup = io < 0
    for k in range(1, 16):
        sh = lax.gather(
            c16, jnp.minimum(io + k, 15)[:, None], dimension_numbers=dn,
            slice_sizes=(1,), mode=lax.GatherScatterMode.PROMISE_IN_BOUNDS)
        dup = dup | ((sh == c16) & (io < 16 - k))
    plsc.store_scatter(win_ref, [c16], eid16, mask=jnp.logical_not(dup))


def _sc_segment(x, row, col):
    mesh = plsc.VectorSubcoreMesh(core_axis_name="c", subcore_axis_name="s")

    @functools.partial(
        pl.kernel,
        mesh=mesh,
        out_type=[
            jax.ShapeDtypeStruct((NC * NPAD, D), jnp.float32),
            jax.ShapeDtypeStruct((NC * NPAD,), jnp.float32),
            jax.ShapeDtypeStruct((NW * NPAD,), jnp.int32),
        ],
        scratch_types=[
            pltpu.VMEM_SHARED((NPAD, D), jnp.float32),
            pltpu.VMEM_SHARED((NPAD,), jnp.float32),
            pltpu.VMEM((KC * SB,), jnp.int32),
            pltpu.VMEM((KC * SB,), jnp.int32),
            pltpu.VMEM((KC, SB), jnp.int32),
            pltpu.VMEM((SB, D), jnp.float32),
            pltpu.VMEM((SB, D), jnp.float32),
            pltpu.VMEM((SB,), jnp.float32),
            pltpu.VMEM((NPAD,), jnp.int32),
            pltpu.VMEM((TROWS,), jnp.float32),
            pltpu.SemaphoreType.DMA,
            pltpu.SemaphoreType.DMA,
        ],
        compiler_params=pltpu.CompilerParams(needs_layout_passes=False),
    )
    def sc_a(x_hbm, row_hbm, col_hbm, sums_hbm, cnts_hbm, win32_hbm,
             accum, cnt_sh, rowf, colf, colc, rb0, rb1,
             ones, win_t, zbuf, sem0, sem1):
        c = lax.axis_index("c")
        s = lax.axis_index("s")
        wid = c * NS + s
        rbufs = [rb0, rb1]
        sems = [sem0, sem1]
        zero16 = jnp.zeros((16,), jnp.float32)

        def zrow(r, _):
            for j in range(D // 16):
                rb0[r, pl.ds(j * 16, 16)] = zero16
            return 0
        lax.fori_loop(0, SB, zrow, 0)

        def zsmall(i, _):
            ones[pl.ds(i * 16, 16)] = zero16 + 1.0
            return 0
        lax.fori_loop(0, SB // 16, zsmall, 0)

        def zzb(i, _):
            zbuf[pl.ds(i * 16, 16)] = zero16
            return 0
        lax.fori_loop(0, TROWS // 16, zzb, 0)

        neg1 = jnp.full((16,), -1, jnp.int32)

        def zwin(i, _):
            win_t[pl.ds(i * 16, 16)] = neg1
            return 0
        lax.fori_loop(0, NPAD // 16, zwin, 0)

        nbase = s * TROWS
        for q in range(TROWS // SB):
            pltpu.sync_copy(rb0, accum.at[pl.ds(nbase + q * SB, SB)])
        pltpu.sync_copy(zbuf, cnt_sh.at[pl.ds(nbase, TROWS)])
        plsc.subcore_barrier()

        cs = wid * NCHT // NW
        cntc = (wid + 1) * NCHT // NW - cs

        def chunk(ch, _):
            @pl.when(ch < cntc)
            def _():
                gc = pl.multiple_of((cs + ch) * KC, 8)
                eo = pl.multiple_of((cs + ch) * KC * SB, 8)

                @pl.when(gc + KC <= NG)
                def _():
                    pltpu.sync_copy(row_hbm.at[pl.ds(eo, KC * SB)], rowf)
                    pltpu.sync_copy(col_hbm.at[pl.ds(eo, KC * SB)], colf)

                @pl.when(gc + KC > NG)
                def _():
                    pltpu.sync_copy(row_hbm.at[pl.ds(eo, REMC * SB)],
                                    rowf.at[pl.ds(0, REMC * SB)])
                    pltpu.sync_copy(col_hbm.at[pl.ds(eo, REMC * SB)],
                                    colf.at[pl.ds(0, REMC * SB)])
                for k in range(KC):
                    for v in range(SB // 16):
                        colc[k, pl.ds(v * 16, 16)] = (
                            colf[pl.ds(k * SB + v * 16, 16)])
                for b in range(NBUF):
                    @pl.when(gc + b < NG)
                    def _():
                        pltpu.async_copy(
                            x_hbm.at[rowf.at[pl.ds(b * SB, SB)]],
                            rbufs[b], sems[b])
                for b in range(KC):
                    r = b % NBUF

                    @pl.when(gc + b < NG)
                    def _():
                        pltpu.make_async_copy(
                            x_hbm.at[rowf.at[pl.ds(b * SB, SB)]],
                            rbufs[r], sems[r]).wait()
                        pltpu.sync_copy(rbufs[r], accum.at[colc.at[b]],
                                        add=True)
                        pltpu.sync_copy(ones, cnt_sh.at[colc.at[b]],
                                        add=True)
                        ebase = (gc + b) * SB
                        for v in range(SB // 16):
                            c16 = colc[b, pl.ds(v * 16, 16)]
                            eid16 = ebase + v * 16 + lax.iota(jnp.int32, 16)
                            _winner_update(win_t, c16, eid16)
                        if b + NBUF < KC:
                            @pl.when(gc + b + NBUF < NG)
                            def _():
                                pltpu.async_copy(
                                    x_hbm.at[rowf.at[pl.ds((b + NBUF) * SB,
                                                           SB)]],
                                    rbufs[r], sems[r])
            return 0
        lax.fori_loop(0, NCHUNK, chunk, 0)

        plsc.subcore_barrier()
        pltpu.sync_copy(win_t,
                        win32_hbm.at[pl.ds(pl.multiple_of(wid * NPAD, 8),
                                           NPAD)])
        hb = pl.multiple_of(c * NPAD + nbase, 8)
        pltpu.sync_copy(accum.at[pl.ds(nbase, TROWS)],
                        sums_hbm.at[pl.ds(hb, TROWS)])
        pltpu.sync_copy(cnt_sh.at[pl.ds(nbase, TROWS)],
                        cnts_hbm.at[pl.ds(hb, TROWS)])

    return sc_a(x, row, col)


def _tc_winmax(win32):
    def body(w_ref, o_ref):
        o_ref[...] = jnp.max(w_ref[...], axis=0, keepdims=True)

    return pl.pallas_call(
        body,
        out_shape=jax.ShapeDtypeStruct((1, NPAD), jnp.int32),
    )(win32)


def _sc_winner(winf, ea_wide):
    mesh = plsc.VectorSubcoreMesh(core_axis_name="c", subcore_axis_name="s")
    NB = NPAD // SB

    @functools.partial(
        pl.kernel,
        mesh=mesh,
        out_type=jax.ShapeDtypeStruct((NPAD, D), jnp.float32),
        scratch_types=[
            pltpu.VMEM((SB,), jnp.int32),
            pltpu.VMEM((SB,), jnp.int32),
            pltpu.VMEM((SB, D), jnp.float32),
            pltpu.SemaphoreType.DMA,
        ],
        compiler_params=pltpu.CompilerParams(needs_layout_passes=False),
    )
    def sc_b(win_hbm, ea_hbm, eaw_hbm, w0, idxb, rowsb, sem):
        c = lax.axis_index("c")
        s = lax.axis_index("s")
        wid = c * NS + s
        bst = NB * wid // NW
        bcnt = NB * (wid + 1) // NW - bst
        for t in range(3):
            b = bst + t

            @pl.when(t < bcnt)
            def _():
                nb = pl.multiple_of(b * SB, 8)
                pltpu.sync_copy(win_hbm.at[pl.ds(nb, SB)], w0)

                def mx(v, _):
                    sl = pl.ds(v * 16, 16)
                    cl = jnp.clip(w0[sl], 0, E - 1)
                    idxb[sl] = lax.shift_right_logical(cl, 3)
                    return 0
                lax.fori_loop(0, SB // 16, mx, 0)
                pltpu.async_copy(ea_hbm.at[idxb], rowsb, sem).wait()
                pltpu.sync_copy(rowsb, eaw_hbm.at[pl.ds(nb, SB)])

    return sc_b(winf, ea_wide)


BK = 1024
NBLK = NPAD // BK


def _tc_dense(sa, sb, ca, cb, win, eaw, x, W_l, b_l, W_r, edge_W, edge_b,
              att_W, att_b, bn_gamma, bn_beta):
    dn = (((1,), (1,)), ((), ()))

    def body1(sa_ref, sb_ref, ca_ref, cb_ref, win_ref, eaw_ref, x_ref,
              wl_ref, bl_ref, wr_ref, ew_ref, eb_ref, aw_ref, ab_ref,
              pre_ref, ps_ref, pq_ref):
        i = pl.program_id(0)
        summed = sa_ref[...] + sb_ref[...]
        counts = ca_ref[...] + cb_ref[...]
        mean = summed / jnp.clip(counts, 1.0, None)
        out = (lax.dot_general(mean, wl_ref[...], dn,
                               preferred_element_type=jnp.float32)
               + lax.dot_general(x_ref[...], wr_ref[...], dn,
                                 preferred_element_type=jnp.float32)
               + bl_ref[...])
        winv = win_ref[...]
        woff = jnp.clip(winv, 0, E - 1) & 7
        ea = jnp.zeros((BK, DE), jnp.float32)
        for k in range(8):
            ea = ea + jnp.where(woff == k,
                                eaw_ref[:, k * DE:(k + 1) * DE], 0.0)
        edge_t = (lax.dot_general(ea, ew_ref[...], dn,
                                  preferred_element_type=jnp.float32)
                  + eb_ref[...])
        a1 = aw_ref[:, 0:D]
        a2 = aw_ref[:, D:2 * D]
        logit = (lax.dot_general(out, a1, dn,
                                 preferred_element_type=jnp.float32)
                 + lax.dot_general(edge_t, a2, dn,
                                   preferred_element_type=jnp.float32)
                 + ab_ref[...])
        att = jax.nn.sigmoid(logit)
        out = out + jnp.where(winv >= 0, att * edge_t, 0.0)
        pre_ref[...] = out
        gid = i * BK + lax.broadcasted_iota(jnp.int32, (BK, 1), 0)
        outm = jnp.where(gid < N, out, 0.0)
        ps_ref[...] = jnp.sum(outm, axis=0, keepdims=True).reshape(1, 1, D)
        pq_ref[...] = jnp.sum(outm * outm, axis=0,
                              keepdims=True).reshape(1, 1, D)

    row_blk = pl.BlockSpec((BK, D), lambda i: (i, 0))
    col_blk = pl.BlockSpec((BK, 1), lambda i: (i, 0))
    row_blk2 = pl.BlockSpec((BK, D), lambda i: (i + NBLK, 0))
    col_blk2 = pl.BlockSpec((BK, 1), lambda i: (i + NBLK, 0))
    full = lambda r, c: pl.BlockSpec((r, c), lambda i: (0, 0))
    pre, ps, pq = pl.pallas_call(
        body1,
        grid=(NBLK,),
        in_specs=[row_blk, row_blk2, col_blk, col_blk2, col_blk, row_blk,
                  row_blk, full(D, D), full(1, D), full(D, D), full(D, DE),
                  full(1, D), full(1, 2 * D), full(1, 1)],
        out_specs=[row_blk, pl.BlockSpec((1, 1, D), lambda i: (i, 0, 0)),
                   pl.BlockSpec((1, 1, D), lambda i: (i, 0, 0))],
        out_shape=[jax.ShapeDtypeStruct((NPAD, D), jnp.float32),
                   jax.ShapeDtypeStruct((NBLK, 1, D), jnp.float32),
                   jax.ShapeDtypeStruct((NBLK, 1, D), jnp.float32)],
    )(sa, sb, ca, cb, win, eaw, x, W_l, b_l, W_r, edge_W, edge_b,
      att_W, att_b)

    def body2(pre_ref, ps_ref, pq_ref, g_ref, be_ref, o_ref):
        s = jnp.sum(ps_ref[...], axis=0)
        q = jnp.sum(pq_ref[...], axis=0)
        mu = s * (1.0 / N)
        var = q * (1.0 / N) - mu * mu
        out = ((pre_ref[...] - mu) * lax.rsqrt(var + 1e-5) * g_ref[...]
               + be_ref[...])
        o_ref[...] = jnp.maximum(out + out, 0.0)

    return pl.pallas_call(
        body2,
        grid=(NBLK,),
        in_specs=[row_blk,
                  pl.BlockSpec((NBLK, 1, D), lambda i: (0, 0, 0)),
                  pl.BlockSpec((NBLK, 1, D), lambda i: (0, 0, 0)),
                  full(1, D), full(1, D)],
        out_specs=row_blk,
        out_shape=jax.ShapeDtypeStruct((NPAD, D), jnp.float32),
    )(pre, ps, pq, bn_gamma, bn_beta)


def kernel(x, edge_index, edge_attr, W_l, b_l, W_r, edge_W, edge_b,
           att_W, att_b, bn_gamma, bn_beta):
    row = edge_index[0]
    col = edge_index[1]
    sums2, cnts2, win32 = _sc_segment(x, row, col)
    winf = _tc_winmax(win32.reshape(NW, NPAD)).reshape(NPAD)
    eaw = _sc_winner(winf, edge_attr.reshape(E // 8, 8 * DE))
    xpad = jnp.pad(x, ((0, NPAD - N), (0, 0)))
    cnts2d = cnts2.reshape(NC * NPAD, 1)
    out = _tc_dense(
        sums2,
        sums2,
        cnts2d,
        cnts2d,
        winf.reshape(NPAD, 1),
        eaw,
        xpad,
        W_l,
        b_l.reshape(1, D),
        W_r,
        edge_W,
        edge_b.reshape(1, D),
        att_W,
        att_b.reshape(1, 1),
        bn_gamma.reshape(1, D),
        bn_beta.reshape(1, D),
    )
    return out[:N]


# speedup vs baseline: 1.0232x; 1.0003x over previous
import functools

import jax
import jax.numpy as jnp
from jax import lax
from jax.experimental import pallas as pl
from jax.experimental.pallas import tpu as pltpu
from jax.experimental.pallas import tpu_sc as plsc

N = 10000
E = 320000
D = 128
DE = 16

NC = 2
NS = 16
NW = NC * NS
NPAD = 10240
TROWS = NPAD // NS
SB = 128
NG = E // SB
KC = 8
NCHT = (NG + KC - 1) // KC
NCHUNK = NCHT // NW + 1
NBUF = 2
REMC = NG % KC


def _winner_update(win_ref, c16, eid16):
    io = lax.iota(jnp.int32, 16)
    dn = lax.GatherDimensionNumbers(
        offset_dims=(), collapsed_slice_dims=(0,), start_index_map=(0,))
    dup = io < 0
    for k in range(1, 16):
        sh = lax.gather(
            c16, jnp.minimum(io + k, 15)[:, None], dimension_numbers=dn,
            slice_sizes=(1,), mode=lax.GatherScatterMode.PROMISE_IN_BOUNDS)
        dup = dup | ((sh == c16) & (io < 16 - k))
    plsc.store_scatter(win_ref, [c16], eid16, mask=jnp.logical_not(dup))


def _sc_segment(x, ei):
    mesh = plsc.VectorSubcoreMesh(core_axis_name="c", subcore_axis_name="s")

    @functools.partial(
        pl.kernel,
        mesh=mesh,
        out_type=[
            jax.ShapeDtypeStruct((NC * NPAD, D), jnp.float32),
            jax.ShapeDtypeStruct((NC * NPAD,), jnp.float32),
            jax.ShapeDtypeStruct((NW * NPAD,), jnp.int32),
        ],
        scratch_types=[
            pltpu.VMEM_SHARED((NPAD, D), jnp.float32),
            pltpu.VMEM_SHARED((NPAD,), jnp.float32),
            pltpu.VMEM((KC * SB,), jnp.int32),
            pltpu.VMEM((KC * SB,), jnp.int32),
            pltpu.VMEM((KC, SB), jnp.int32),
            pltpu.VMEM((SB, D), jnp.float32),
            pltpu.VMEM((SB, D), jnp.float32),
            pltpu.VMEM((SB,), jnp.float32),
            pltpu.VMEM((NPAD,), jnp.int32),
            pltpu.VMEM((TROWS,), jnp.float32),
            pltpu.SemaphoreType.DMA,
            pltpu.SemaphoreType.DMA,
        ],
        compiler_params=pltpu.CompilerParams(needs_layout_passes=False),
    )
    def sc_a(x_hbm, ei_hbm, sums_hbm, cnts_hbm, win32_hbm,
             accum, cnt_sh, rowf, colf, colc, rb0, rb1,
             ones, win_t, zbuf, sem0, sem1):
        c = lax.axis_index("c")
        s = lax.axis_index("s")
        wid = c * NS + s
        rbufs = [rb0, rb1]
        sems = [sem0, sem1]
        zero16 = jnp.zeros((16,), jnp.float32)

        def zrow(r, _):
            for j in range(D // 16):
                rb0[r, pl.ds(j * 16, 16)] = zero16
            return 0
        lax.fori_loop(0, SB, zrow, 0)

        def zsmall(i, _):
            ones[pl.ds(i * 16, 16)] = zero16 + 1.0
            return 0
        lax.fori_loop(0, SB // 16, zsmall, 0)

        def zzb(i, _):
            zbuf[pl.ds(i * 16, 16)] = zero16
            return 0
        lax.fori_loop(0, TROWS // 16, zzb, 0)

        neg1 = jnp.full((16,), -1, jnp.int32)

        def zwin(i, _):
            win_t[pl.ds(i * 16, 16)] = neg1
            return 0
        lax.fori_loop(0, NPAD // 16, zwin, 0)

        nbase = s * TROWS
        for q in range(TROWS // SB):
            pltpu.sync_copy(rb0, accum.at[pl.ds(nbase + q * SB, SB)])
        pltpu.sync_copy(zbuf, cnt_sh.at[pl.ds(nbase, TROWS)])
        plsc.subcore_barrier()

        cs = wid * NCHT // NW
        cntc = (wid + 1) * NCHT // NW - cs

        def chunk(ch, _):
            @pl.when(ch < cntc)
            def _():
                gc = pl.multiple_of((cs + ch) * KC, 8)
                eo = pl.multiple_of((cs + ch) * KC * SB, 8)

                co = pl.multiple_of(E + (cs + ch) * KC * SB, 8)

                @pl.when(gc + KC <= NG)
                def _():
                    pltpu.sync_copy(ei_hbm.at[pl.ds(eo, KC * SB)], rowf)
                    pltpu.sync_copy(ei_hbm.at[pl.ds(co, KC * SB)], colf)

                @pl.when(gc + KC > NG)
                def _():
                    pltpu.sync_copy(ei_hbm.at[pl.ds(eo, REMC * SB)],
                                    rowf.at[pl.ds(0, REMC * SB)])
                    pltpu.sync_copy(ei_hbm.at[pl.ds(co, REMC * SB)],
                                    colf.at[pl.ds(0, REMC * SB)])
                for k in range(KC):
                    for v in range(SB // 16):
                        colc[k, pl.ds(v * 16, 16)] = (
                            colf[pl.ds(k * SB + v * 16, 16)])
                for b in range(NBUF):
                    @pl.when(gc + b < NG)
                    def _():
                        pltpu.async_copy(
                            x_hbm.at[rowf.at[pl.ds(b * SB, SB)]],
                            rbufs[b], sems[b])
                for b in range(KC):
                    r = b % NBUF

                    @pl.when(gc + b < NG)
                    def _():
                        pltpu.make_async_copy(
                            x_hbm.at[rowf.at[pl.ds(b * SB, SB)]],
                            rbufs[r], sems[r]).wait()
                        pltpu.sync_copy(rbufs[r], accum.at[colc.at[b]],
                                        add=True)
                        pltpu.sync_copy(ones, cnt_sh.at[colc.at[b]],
                                        add=True)
                        ebase = (gc + b) * SB
                        for v in range(SB // 16):
                            c16 = colc[b, pl.ds(v * 16, 16)]
                            eid16 = ebase + v * 16 + lax.iota(jnp.int32, 16)
                            _winner_update(win_t, c16, eid16)
                        if b + NBUF < KC:
                            @pl.when(gc + b + NBUF < NG)
                            def _():
                                pltpu.async_copy(
                                    x_hbm.at[rowf.at[pl.ds((b + NBUF) * SB,
                                                           SB)]],
                                    rbufs[r], sems[r])
            return 0
        lax.fori_loop(0, NCHUNK, chunk, 0)

        plsc.subcore_barrier()
        pltpu.sync_copy(win_t,
                        win32_hbm.at[pl.ds(pl.multiple_of(wid * NPAD, 8),
                                           NPAD)])
        hb = pl.multiple_of(c * NPAD + nbase, 8)
        pltpu.sync_copy(accum.at[pl.ds(nbase, TROWS)],
                        sums_hbm.at[pl.ds(hb, TROWS)])
        pltpu.sync_copy(cnt_sh.at[pl.ds(nbase, TROWS)],
                        cnts_hbm.at[pl.ds(hb, TROWS)])

    return sc_a(x, ei)


def _tc_winmax(win32):
    def body(w_ref, o_ref):
        o_ref[...] = jnp.max(w_ref[...], axis=0, keepdims=True)

    return pl.pallas_call(
        body,
        out_shape=jax.ShapeDtypeStruct((1, NPAD), jnp.int32),
    )(win32)


def _sc_winner(winf, ea_wide):
    mesh = plsc.VectorSubcoreMesh(core_axis_name="c", subcore_axis_name="s")
    NB = NPAD // SB

    @functools.partial(
        pl.kernel,
        mesh=mesh,
        out_type=jax.ShapeDtypeStruct((NPAD, D), jnp.float32),
        scratch_types=[
            pltpu.VMEM((SB,), jnp.int32),
            pltpu.VMEM((SB,), jnp.int32),
            pltpu.VMEM((SB, D), jnp.float32),
            pltpu.SemaphoreType.DMA,
        ],
        compiler_params=pltpu.CompilerParams(needs_layout_passes=False),
    )
    def sc_b(win_hbm, ea_hbm, eaw_hbm, w0, idxb, rowsb, sem):
        c = lax.axis_index("c")
        s = lax.axis_index("s")
        wid = c * NS + s
        bst = NB * wid // NW
        bcnt = NB * (wid + 1) // NW - bst
        for t in range(3):
            b = bst + t

            @pl.when(t < bcnt)
            def _():
                nb = pl.multiple_of(b * SB, 8)
                pltpu.sync_copy(win_hbm.at[pl.ds(nb, SB)], w0)

                def mx(v, _):
                    sl = pl.ds(v * 16, 16)
                    cl = jnp.clip(w0[sl], 0, E - 1)
                    idxb[sl] = lax.shift_right_logical(cl, 3)
                    return 0
                lax.fori_loop(0, SB // 16, mx, 0)
                pltpu.async_copy(ea_hbm.at[idxb], rowsb, sem).wait()
                pltpu.sync_copy(rowsb, eaw_hbm.at[pl.ds(nb, SB)])

    return sc_b(winf, ea_wide)


BK = 1024
NBLK = NPAD // BK


def _tc_dense(sa, sb, ca, cb, win, eaw, x, W_l, b_l, W_r, edge_W, edge_b,
              att_W, att_b, bn_gamma, bn_beta):
    dn = (((1,), (1,)), ((), ()))

    def body1(sa_ref, sb_ref, ca_ref, cb_ref, win_ref, eaw_ref, x_ref,
              wl_ref, bl_ref, wr_ref, ew_ref, eb_ref, aw_ref, ab_ref,
              pre_ref, ps_ref, pq_ref):
        i = pl.program_id(0)
        summed = sa_ref[...] + sb_ref[...]
        counts = ca_ref[...] + cb_ref[...]
        mean = summed / jnp.clip(counts, 1.0, None)
        out = (lax.dot_general(mean, wl_ref[...], dn,
                               preferred_element_type=jnp.float32)
               + lax.dot_general(x_ref[...], wr_ref[...], dn,
                                 preferred_element_type=jnp.float32)
               + bl_ref[...])
        winv = win_ref[...]
        woff = jnp.clip(winv, 0, E - 1) & 7
        ea = jnp.zeros((BK, DE), jnp.float32)
        for k in range(8):
            ea = ea + jnp.where(woff == k,
                                eaw_ref[:, k * DE:(k + 1) * DE], 0.0)
        edge_t = (lax.dot_general(ea, ew_ref[...], dn,
                                  preferred_element_type=jnp.float32)
                  + eb_ref[...])
        a1 = aw_ref[:, 0:D]
        a2 = aw_ref[:, D:2 * D]
        logit = (lax.dot_general(out, a1, dn,
                                 preferred_element_type=jnp.float32)
                 + lax.dot_general(edge_t, a2, dn,
                                   preferred_element_type=jnp.float32)
                 + ab_ref[...])
        att = jax.nn.sigmoid(logit)
        out = out + jnp.where(winv >= 0, att * edge_t, 0.0)
        pre_ref[...] = out
        gid = i * BK + lax.broadcasted_iota(jnp.int32, (BK, 1), 0)
        outm = jnp.where(gid < N, out, 0.0)
        ps_ref[...] = jnp.sum(outm, axis=0, keepdims=True).reshape(1, 1, D)
        pq_ref[...] = jnp.sum(outm * outm, axis=0,
                              keepdims=True).reshape(1, 1, D)

    row_blk = pl.BlockSpec((BK, D), lambda i: (i, 0))
    col_blk = pl.BlockSpec((BK, 1), lambda i: (i, 0))
    row_blk2 = pl.BlockSpec((BK, D), lambda i: (i + NBLK, 0))
    col_blk2 = pl.BlockSpec((BK, 1), lambda i: (i + NBLK, 0))
    full = lambda r, c: pl.BlockSpec((r, c), lambda i: (0, 0))
    pre, ps, pq = pl.pallas_call(
        body1,
        grid=(NBLK,),
        in_specs=[row_blk, row_blk2, col_blk, col_blk2, col_blk, row_blk,
                  row_blk, full(D, D), full(1, D), full(D, D), full(D, DE),
                  full(1, D), full(1, 2 * D), full(1, 1)],
        out_specs=[row_blk, pl.BlockSpec((1, 1, D), lambda i: (i, 0, 0)),
                   pl.BlockSpec((1, 1, D), lambda i: (i, 0, 0))],
        out_shape=[jax.ShapeDtypeStruct((NPAD, D), jnp.float32),
                   jax.ShapeDtypeStruct((NBLK, 1, D), jnp.float32),
                   jax.ShapeDtypeStruct((NBLK, 1, D), jnp.float32)],
    )(sa, sb, ca, cb, win, eaw, x, W_l, b_l, W_r, edge_W, edge_b,
      att_W, att_b)

    def body2(pre_ref, ps_ref, pq_ref, g_ref, be_ref, o_ref):
        s = jnp.sum(ps_ref[...], axis=0)
        q = jnp.sum(pq_ref[...], axis=0)
        mu = s * (1.0 / N)
        var = q * (1.0 / N) - mu * mu
        out = ((pre_ref[...] - mu) * lax.rsqrt(var + 1e-5) * g_ref[...]
               + be_ref[...])
        o_ref[...] = jnp.maximum(out + out, 0.0)

    return pl.pallas_call(
        body2,
        grid=(NBLK,),
        in_specs=[row_blk,
                  pl.BlockSpec((NBLK, 1, D), lambda i: (0, 0, 0)),
                  pl.BlockSpec((NBLK, 1, D), lambda i: (0, 0, 0)),
                  full(1, D), full(1, D)],
        out_specs=row_blk,
        out_shape=jax.ShapeDtypeStruct((NPAD, D), jnp.float32),
    )(pre, ps, pq, bn_gamma, bn_beta)


def kernel(x, edge_index, edge_attr, W_l, b_l, W_r, edge_W, edge_b,
           att_W, att_b, bn_gamma, bn_beta):
    sums2, cnts2, win32 = _sc_segment(x, edge_index.reshape(2 * E))
    winf = _tc_winmax(win32.reshape(NW, NPAD)).reshape(NPAD)
    eaw = _sc_winner(winf, edge_attr.reshape(E // 8, 8 * DE))
    xpad = jnp.pad(x, ((0, NPAD - N), (0, 0)))
    cnts2d = cnts2.reshape(NC * NPAD, 1)
    out = _tc_dense(
        sums2,
        sums2,
        cnts2d,
        cnts2d,
        winf.reshape(NPAD, 1),
        eaw,
        xpad,
        W_l,
        b_l.reshape(1, D),
        W_r,
        edge_W,
        edge_b.reshape(1, D),
        att_W,
        att_b.reshape(1, 1),
        bn_gamma.reshape(1, D),
        bn_beta.reshape(1, D),
    )
    return out[:N]
